# baseline (speedup 1.0000x reference)
import functools

import jax
import jax.numpy as jnp
from jax import lax
from jax.experimental import pallas as pl
from jax.experimental.pallas import tpu as pltpu
from jax.experimental.pallas import tpu_sc as plsc


def _argmax_block_body(x_ref, out_ref):
    x = x_ref[...]
    m = jnp.max(x, axis=1, keepdims=True)
    ii = lax.broadcasted_iota(jnp.int32, x.shape, 1)
    cand = jnp.where(x == m, ii, jnp.int32(x.shape[1]))
    out_ref[:, 0] = jnp.min(cand, axis=1)


def _argmax_tc(inputs, block_rows=8, interpret=False):
    b, v = inputs.shape
    assert b % block_rows == 0
    return pl.pallas_call(
        _argmax_block_body,
        grid=(b // block_rows,),
        in_specs=[pl.BlockSpec((block_rows, v), lambda i: (i, 0))],
        out_specs=pl.BlockSpec((block_rows, 1), lambda i: (i, 0)),
        out_shape=jax.ShapeDtypeStruct((b, 1), jnp.int32),
        interpret=interpret,
    )(inputs)


def _gather_sc(embeddings, idx):
    (b,) = idx.shape
    v, d = embeddings.shape
    info = plsc.get_sparse_core_info()
    nw = info.num_cores * info.num_subcores
    assert b % (8 * nw) == 0 and d % info.num_lanes == 0
    b_per_w = b // nw
    mesh = plsc.VectorSubcoreMesh(core_axis_name="c", subcore_axis_name="s")

    @functools.partial(
        pl.kernel,
        mesh=mesh,
        out_type=jax.ShapeDtypeStruct((b, d), jnp.float32),
        scratch_types=[
            pltpu.VMEM((b_per_w,), jnp.int32),
            pltpu.VMEM((b_per_w, d), jnp.float32),
            pltpu.SemaphoreType.DMA,
        ],
        compiler_params=pltpu.CompilerParams(use_tc_tiling_on_sc=False),
    )
    def gather_kernel(table_hbm, idx_hbm, out_hbm, idx_v, rows_v, sem):
        wid = lax.axis_index("s") * info.num_cores + lax.axis_index("c")
        base = wid * b_per_w
        pltpu.sync_copy(idx_hbm.at[pl.ds(base, b_per_w)], idx_v)
        pltpu.async_copy(table_hbm.at[idx_v], rows_v, sem).wait()
        pltpu.sync_copy(rows_v, out_hbm.at[pl.ds(base, b_per_w)])

    return gather_kernel(embeddings, idx)


def kernel(inputs, embeddings):
    idx = _argmax_tc(inputs).reshape(inputs.shape[0])
    return _gather_sc(embeddings, idx)


# speedup vs baseline: 1.0536x; 1.0536x over previous
import functools

import jax
import jax.numpy as jnp
from jax import lax
from jax.experimental import pallas as pl
from jax.experimental.pallas import tpu as pltpu
from jax.experimental.pallas import tpu_sc as plsc


_LANES = 128


def _argmax_block_body(x_ref, out_ref):
    br, v = x_ref.shape
    nfull = v // _LANES
    lane = lax.broadcasted_iota(jnp.int32, (br, _LANES), 1)

    m = x_ref[:, 0:_LANES]
    g = lane
    for j in range(1, nfull):
        base = j * _LANES
        chunk = x_ref[:, base : base + _LANES]
        upd = chunk > m
        m = jnp.where(upd, chunk, m)
        g = jnp.where(upd, lane + base, g)
    if v % _LANES:
        base = v - _LANES
        chunk = x_ref[:, base : base + _LANES]
        upd = chunk > m
        m = jnp.where(upd, chunk, m)
        g = jnp.where(upd, lane + base, g)

    rowmax = jnp.max(m, axis=1, keepdims=True)
    cand = jnp.where(m == rowmax, g, jnp.int32(v))
    out_ref[:, 0] = jnp.min(cand, axis=1)


def _argmax_tc(inputs, block_rows=8, interpret=False):
    b, v = inputs.shape
    assert b % block_rows == 0
    return pl.pallas_call(
        _argmax_block_body,
        grid=(b // block_rows,),
        in_specs=[pl.BlockSpec((block_rows, v), lambda i: (i, 0))],
        out_specs=pl.BlockSpec((block_rows, 1), lambda i: (i, 0)),
        out_shape=jax.ShapeDtypeStruct((b, 1), jnp.int32),
        interpret=interpret,
    )(inputs)


def _gather_sc(embeddings, idx):
    (b,) = idx.shape
    v, d = embeddings.shape
    info = plsc.get_sparse_core_info()
    nw = info.num_cores * info.num_subcores
    assert b % (8 * nw) == 0 and d % info.num_lanes == 0
    b_per_w = b // nw
    mesh = plsc.VectorSubcoreMesh(core_axis_name="c", subcore_axis_name="s")

    @functools.partial(
        pl.kernel,
        mesh=mesh,
        out_type=jax.ShapeDtypeStruct((b, d), jnp.float32),
        scratch_types=[
            pltpu.VMEM((b_per_w,), jnp.int32),
            pltpu.VMEM((b_per_w, d), jnp.float32),
            pltpu.SemaphoreType.DMA,
        ],
        compiler_params=pltpu.CompilerParams(use_tc_tiling_on_sc=False),
    )
    def gather_kernel(table_hbm, idx_hbm, out_hbm, idx_v, rows_v, sem):
        wid = lax.axis_index("s") * info.num_cores + lax.axis_index("c")
        base = wid * b_per_w
        pltpu.sync_copy(idx_hbm.at[pl.ds(base, b_per_w)], idx_v)
        pltpu.async_copy(table_hbm.at[idx_v], rows_v, sem).wait()
        pltpu.sync_copy(rows_v, out_hbm.at[pl.ds(base, b_per_w)])

    return gather_kernel(embeddings, idx)


def kernel(inputs, embeddings):
    idx = _argmax_tc(inputs).reshape(inputs.shape[0])
    return _gather_sc(embeddings, idx)
